# baseline (speedup 1.0000x reference)
import dataclasses

import jax
import jax.numpy as jnp
from jax import lax
from jax.experimental import pallas as pl
from jax.experimental.pallas import tpu as pltpu
from jax.experimental.pallas import tpu_sc as plsc

N = 10000
E = 320000
H = 128
NC = 2
NS = 16
NW = NC * NS
EC = 80
SLABS = 2
ES = E // SLABS
NCHUNK = ES // EC
CPW = -(-NCHUNK // NW)
NPAD = 10240
RPS = NPAD // NS
ZR = 64

RB = 2000
EB = 1280

_f32 = jnp.float32


def _prep_body(x_ref, wemb_ref, bemb_ref, w1a_ref, w1b_ref, b1_ref,
               h_ref, t1_ref, t2_ref):
    xb = x_ref[...]
    h = jnp.dot(xb, wemb_ref[...], preferred_element_type=_f32) + bemb_ref[...]
    h_ref[...] = h
    t1_ref[...] = jnp.dot(h, w1a_ref[...], preferred_element_type=_f32)
    t2_ref[...] = jnp.dot(h, w1b_ref[...], preferred_element_type=_f32) + b1_ref[...]


def _edge_body(g1_ref, g2_ref, d2_ref, w1dmat_ref, w2_ref, b2_ref, wimat_ref,
               bi_ref, msg_ref):
    s = jnp.sqrt(d2_ref[0])
    r_i = jax.lax.broadcasted_iota(jnp.int32, (EB, EB // H), 0)
    c_i = jax.lax.broadcasted_iota(jnp.int32, (EB, EB // H), 1)
    rep = (c_i == r_i // H).astype(_f32)
    drows = jnp.dot(rep, s, preferred_element_type=_f32)
    r2 = jax.lax.broadcasted_iota(jnp.int32, (EB, H), 0)
    j2 = jax.lax.broadcasted_iota(jnp.int32, (EB, H), 1)
    dsel = jnp.where(j2 == r2 % H, drows, 0.0)
    distw = jnp.dot(dsel, w1dmat_ref[...], preferred_element_type=_f32)
    z1 = g1_ref[...] + g2_ref[...] + distw
    z1b = z1.astype(jnp.bfloat16)
    u = z1b * jax.nn.sigmoid(z1b)
    v0 = (jnp.dot(u, w2_ref[...].astype(jnp.bfloat16),
                  preferred_element_type=_f32) + b2_ref[...])
    v0b = v0.astype(jnp.bfloat16)
    v = v0b * jax.nn.sigmoid(v0b)
    p = (jnp.dot(v, wimat_ref[...].astype(jnp.bfloat16),
                 preferred_element_type=_f32) + bi_ref[...]).astype(jnp.bfloat16)
    gate = jax.nn.sigmoid(p)
    msg_ref[...] = (gate * v).astype(_f32)


def _node_body(h_ref, mia_ref, mib_ref, wh1a_ref, wh1b_ref, bh1_ref, wh2_ref,
               bh2_ref, wl1_ref, bl1_ref, wl2_ref, bl2_ref, out_ref):
    h = h_ref[...]
    m = (mia_ref[0] + mia_ref[1]) + (mib_ref[0] + mib_ref[1])
    t0 = (jnp.dot(h, wh1a_ref[...], preferred_element_type=_f32)
          + jnp.dot(m, wh1b_ref[...], preferred_element_type=_f32)
          + bh1_ref[...])
    t = t0 * jax.nn.sigmoid(t0)
    h2 = h + jnp.dot(t, wh2_ref[...], preferred_element_type=_f32) + bh2_ref[...]
    z = jnp.tanh(jnp.dot(h2, wl1_ref[...], preferred_element_type=_f32) + bl1_ref[...])
    out_ref[...] = jnp.sum(z * wl2_ref[...], axis=1, keepdims=True) + bl2_ref[...]


GSLOT = 4
SSLOT = 3


def _gather_sc(t1_hbm, t2_hbm, src_hbm, dst_hbm, px_hbm, py_hbm,
               g1_hbm, g2_hbm, d2_hbm,
               pxv, pyv, *slot_refs):
    w = lax.axis_index("s") * NC + lax.axis_index("c")
    pltpu.sync_copy(px_hbm, pxv)
    pltpu.sync_copy(py_hbm, pyv)
    idxs = slot_refs[0:GSLOT]
    idxd = slot_refs[GSLOT:2 * GSLOT]
    b1 = slot_refs[2 * GSLOT:3 * GSLOT]
    b2 = slot_refs[3 * GSLOT:4 * GSLOT]
    d2b = slot_refs[4 * GSLOT:5 * GSLOT]
    sg = slot_refs[5 * GSLOT:6 * GSLOT]
    sw = slot_refs[6 * GSLOT:7 * GSLOT]

    def base_of(k):
        return (k * NW + w) * EC

    def load_idx(k, b):
        base = base_of(k)
        pltpu.sync_copy(src_hbm.at[pl.ds(base, EC)], idxs[b])
        pltpu.sync_copy(dst_hbm.at[pl.ds(base, EC)], idxd[b])

    def gather_copies(b):
        return (pltpu.make_async_copy(t1_hbm.at[idxs[b]], b1[b], sg[b]),
                pltpu.make_async_copy(t2_hbm.at[idxd[b]], b2[b], sg[b]))

    def wb_copies(k, b):
        base = base_of(k)
        return (pltpu.make_async_copy(b1[b], g1_hbm.at[pl.ds(base, EC)], sw[b]),
                pltpu.make_async_copy(b2[b], g2_hbm.at[pl.ds(base, EC)], sw[b]),
                pltpu.make_async_copy(d2b[b], d2_hbm.at[pl.ds(base, EC)], sw[b]))

    def compute_d2(b):
        @pl.loop(0, EC, step=16)
        def _(i):
            i16s = idxs[b][pl.ds(i, 16)]
            i16d = idxd[b][pl.ds(i, 16)]
            dx = plsc.load_gather(pxv, [i16s]) - plsc.load_gather(pxv, [i16d])
            dy = plsc.load_gather(pyv, [i16s]) - plsc.load_gather(pyv, [i16d])
            d2b[b][pl.ds(i, 16)] = dx * dx + dy * dy

    for b in range(GSLOT):
        load_idx(b, b)
        for cp in gather_copies(b):
            cp.start()

    @pl.loop(0, CPW, step=GSLOT)
    def _(k0):
        for b in range(GSLOT):
            k = k0 + b

            @pl.when(k * NW + w < NCHUNK)
            def _(k=k, b=b):
                for cp in gather_copies(b):
                    cp.wait()
                compute_d2(b)
                for cp in wb_copies(k, b):
                    cp.start()

                @pl.when((k + GSLOT) * NW + w < NCHUNK)
                def _():
                    load_idx(k + GSLOT, b)
                    for cp in wb_copies(k, b):
                        cp.wait()
                    for cp in gather_copies(b):
                        cp.start()

                @pl.when((k + GSLOT) * NW + w >= NCHUNK)
                def _():
                    for cp in wb_copies(k, b):
                        cp.wait()


def _scatter_sc(msg_hbm, src_hbm, mi_hbm, zbuf, acc, *slot_refs):
    c = lax.axis_index("c")
    s = lax.axis_index("s")
    w = s * NC + c
    idx = slot_refs[0:SSLOT]
    mb = slot_refs[SSLOT:2 * SSLOT]
    sl = slot_refs[2 * SSLOT:3 * SSLOT]

    z16 = jnp.zeros((16,), _f32)

    @pl.loop(0, ZR)
    def _(r):
        @pl.loop(0, H, step=16)
        def _(col):
            zbuf[r, pl.ds(col, 16)] = z16

    @pl.loop(0, RPS // ZR)
    def _(i):
        pltpu.sync_copy(zbuf, acc.at[pl.ds(s * RPS + i * ZR, ZR)])

    plsc.subcore_barrier()

    def base_of(k):
        return (k * NW + w) * EC

    def load_copies(k, b):
        base = base_of(k)
        return (pltpu.make_async_copy(src_hbm.at[pl.ds(base, EC)], idx[b], sl[b]),
                pltpu.make_async_copy(msg_hbm.at[pl.ds(base, EC)], mb[b], sl[b]))

    for b in range(SSLOT):
        for cp in load_copies(b, b):
            cp.start()

    @pl.loop(0, CPW, step=SSLOT)
    def _(k0):
        for b in range(SSLOT):
            k = k0 + b

            @pl.when(k * NW + w < NCHUNK)
            def _(k=k, b=b):
                for cp in load_copies(k, b):
                    cp.wait()
                pltpu.sync_copy(mb[b], acc.at[idx[b]], add=True)

                @pl.when((k + SSLOT) * NW + w < NCHUNK)
                def _():
                    for cp in load_copies(k + SSLOT, b):
                        cp.start()

    plsc.subcore_barrier()
    pltpu.sync_copy(acc.at[pl.ds(s * RPS, RPS)], mi_hbm.at[c, pl.ds(s * RPS, RPS)])


def _full(shape):
    return pl.BlockSpec(shape, lambda i: tuple(0 for _ in shape))


def kernel(x, edge_index, W_emb, b_emb, W1, b1, W2, b2, Wi, bi,
           Wh1, bh1, Wh2, bh2, Wl1, bl1, Wl2, bl2):
    x = x.astype(_f32)
    src = edge_index[0].astype(jnp.int32)
    dst = edge_index[1].astype(jnp.int32)
    px = x[:, 0]
    py = x[:, 1]
    _sc_mesh = plsc.VectorSubcoreMesh(core_axis_name="c", subcore_axis_name="s")
    _sc_cp = pltpu.CompilerParams()
    if "needs_layout_passes" in pltpu.CompilerParams.__dataclass_fields__:
        _sc_cp = dataclasses.replace(_sc_cp, needs_layout_passes=False)

    h, t1, t2 = pl.pallas_call(
        _prep_body,
        grid=(N // RB,),
        in_specs=[
            pl.BlockSpec((RB, H), lambda i: (i, 0)),
            _full((H, H)), _full((1, H)), _full((H, H)), _full((H, H)), _full((1, H)),
        ],
        out_specs=[
            pl.BlockSpec((RB, H), lambda i: (i, 0)),
            pl.BlockSpec((RB, H), lambda i: (i, 0)),
            pl.BlockSpec((RB, H), lambda i: (i, 0)),
        ],
        out_shape=[
            jax.ShapeDtypeStruct((N, H), _f32),
            jax.ShapeDtypeStruct((N, H), _f32),
            jax.ShapeDtypeStruct((N, H), _f32),
        ],
    )(x, W_emb, b_emb.reshape(1, H), W1[:H], W1[H:2 * H], b1.reshape(1, H))

    gather = pl.kernel(
        _gather_sc,
        out_type=[
            jax.ShapeDtypeStruct((ES, H), _f32),
            jax.ShapeDtypeStruct((ES, H), _f32),
            jax.ShapeDtypeStruct((ES,), _f32),
        ],
        mesh=_sc_mesh,
        scratch_types=(
            [pltpu.VMEM((N,), _f32), pltpu.VMEM((N,), _f32)]
            + [pltpu.VMEM((EC,), jnp.int32)] * GSLOT
            + [pltpu.VMEM((EC,), jnp.int32)] * GSLOT
            + [pltpu.VMEM((EC, H), _f32)] * GSLOT
            + [pltpu.VMEM((EC, H), _f32)] * GSLOT
            + [pltpu.VMEM((EC,), _f32)] * GSLOT
            + [pltpu.SemaphoreType.DMA] * (2 * GSLOT)
        ),
        compiler_params=_sc_cp,
    )

    scatter = pl.kernel(
        _scatter_sc,
        out_type=jax.ShapeDtypeStruct((NC, NPAD, H), _f32),
        mesh=_sc_mesh,
        scratch_types=(
            [pltpu.VMEM((ZR, H), _f32), pltpu.VMEM_SHARED((NPAD, H), _f32)]
            + [pltpu.VMEM((EC,), jnp.int32)] * SSLOT
            + [pltpu.VMEM((EC, H), _f32)] * SSLOT
            + [pltpu.SemaphoreType.DMA] * SSLOT
        ),
    )

    w1dmat = jnp.tile(W1[2 * H].reshape(1, H), (H, 1))
    wimat = jnp.tile(Wi.reshape(H, 1), (1, H))
    mi_parts = []
    for sidx in range(SLABS):
        src_s = lax.slice_in_dim(src, sidx * ES, (sidx + 1) * ES)
        dst_s = lax.slice_in_dim(dst, sidx * ES, (sidx + 1) * ES)
        g1, g2, d2 = gather(t1, t2, src_s, dst_s, px, py)
        msg = pl.pallas_call(
            _edge_body,
            grid=(ES // EB,),
            in_specs=[
                pl.BlockSpec((EB, H), lambda i: (i, 0)),
                pl.BlockSpec((EB, H), lambda i: (i, 0)),
                pl.BlockSpec((1, EB // H, H), lambda i: (i, 0, 0)),
                _full((H, H)), _full((H, H)), _full((1, H)), _full((H, H)), _full((1, 1)),
            ],
            out_specs=pl.BlockSpec((EB, H), lambda i: (i, 0)),
            out_shape=jax.ShapeDtypeStruct((ES, H), _f32),
        )(g1, g2, d2.reshape(ES // EB, EB // H, H),
          w1dmat, W2, b2.reshape(1, H), wimat, bi.reshape(1, 1))
        mi_parts.append(scatter(msg, src_s))

    out = pl.pallas_call(
        _node_body,
        grid=(N // RB,),
        in_specs=[
            pl.BlockSpec((RB, H), lambda i: (i, 0)),
            pl.BlockSpec((NC, RB, H), lambda i: (0, i, 0)),
            pl.BlockSpec((NC, RB, H), lambda i: (0, i, 0)),
            _full((H, H)), _full((H, H)), _full((1, H)),
            _full((H, H)), _full((1, H)),
            _full((H, H)), _full((1, H)), _full((1, H)), _full((1, 1)),
        ],
        out_specs=pl.BlockSpec((RB, 1), lambda i: (i, 0)),
        out_shape=jax.ShapeDtypeStruct((N, 1), _f32),
    )(h, mi_parts[0], mi_parts[1], Wh1[:H], Wh1[H:], bh1.reshape(1, H),
      Wh2, bh2.reshape(1, H), Wl1, bl1.reshape(1, H), Wl2.reshape(1, H),
      bl2.reshape(1, 1))

    return out


# speedup vs baseline: 1.0327x; 1.0327x over previous
import dataclasses

import jax
import jax.numpy as jnp
from jax import lax
from jax.experimental import pallas as pl
from jax.experimental.pallas import tpu as pltpu
from jax.experimental.pallas import tpu_sc as plsc

N = 10000
E = 320000
H = 128
NC = 2
NS = 16
NW = NC * NS
EC = 80
SLABS = 1
ES = E // SLABS
NCHUNK = ES // EC
CPW = -(-NCHUNK // NW)
NPAD = 10240
RPS = NPAD // NS
ZR = 64

RB = 2000
EB = 3200

_f32 = jnp.float32


def _prep_body(x_ref, wemb_ref, bemb_ref, w1a_ref, w1b_ref, b1_ref,
               h_ref, t1_ref, t2_ref):
    xb = x_ref[...]
    h = jnp.dot(xb, wemb_ref[...], preferred_element_type=_f32) + bemb_ref[...]
    h_ref[...] = h
    t1_ref[...] = jnp.dot(h, w1a_ref[...], preferred_element_type=_f32)
    t2_ref[...] = jnp.dot(h, w1b_ref[...], preferred_element_type=_f32) + b1_ref[...]


def _edge_body(g1_ref, g2_ref, d2_ref, w1dmat_ref, w2_ref, b2_ref, wimat_ref,
               bi_ref, msg_ref):
    s = jnp.sqrt(d2_ref[0])
    r_i = jax.lax.broadcasted_iota(jnp.int32, (EB, EB // H), 0)
    c_i = jax.lax.broadcasted_iota(jnp.int32, (EB, EB // H), 1)
    rep = (c_i == r_i // H).astype(_f32)
    drows = jnp.dot(rep, s, preferred_element_type=_f32)
    r2 = jax.lax.broadcasted_iota(jnp.int32, (EB, H), 0)
    j2 = jax.lax.broadcasted_iota(jnp.int32, (EB, H), 1)
    dsel = jnp.where(j2 == r2 % H, drows, 0.0)
    distw = jnp.dot(dsel, w1dmat_ref[...], preferred_element_type=_f32)
    z1 = g1_ref[...] + g2_ref[...] + distw
    z1b = z1.astype(jnp.bfloat16)
    u = z1b * jax.nn.sigmoid(z1b)
    v0 = (jnp.dot(u, w2_ref[...].astype(jnp.bfloat16),
                  preferred_element_type=_f32) + b2_ref[...])
    v0b = v0.astype(jnp.bfloat16)
    v = v0b * jax.nn.sigmoid(v0b)
    p = (jnp.dot(v, wimat_ref[...].astype(jnp.bfloat16),
                 preferred_element_type=_f32) + bi_ref[...]).astype(jnp.bfloat16)
    gate = jax.nn.sigmoid(p)
    msg_ref[...] = (gate * v).astype(_f32)


def _node_body(h_ref, *refs):
    mi_refs = refs[:SLABS]
    (wh1a_ref, wh1b_ref, bh1_ref, wh2_ref, bh2_ref,
     wl1_ref, bl1_ref, wl2_ref, bl2_ref, out_ref) = refs[SLABS:]
    h = h_ref[...]
    m = mi_refs[0][0] + mi_refs[0][1]
    for r in mi_refs[1:]:
        m = m + r[0] + r[1]
    t0 = (jnp.dot(h, wh1a_ref[...], preferred_element_type=_f32)
          + jnp.dot(m, wh1b_ref[...], preferred_element_type=_f32)
          + bh1_ref[...])
    t = t0 * jax.nn.sigmoid(t0)
    h2 = h + jnp.dot(t, wh2_ref[...], preferred_element_type=_f32) + bh2_ref[...]
    z = jnp.tanh(jnp.dot(h2, wl1_ref[...], preferred_element_type=_f32) + bl1_ref[...])
    out_ref[...] = jnp.sum(z * wl2_ref[...], axis=1, keepdims=True) + bl2_ref[...]


GSLOT = 4
SSLOT = 3


def _gather_sc(t1_hbm, t2_hbm, src_hbm, dst_hbm, px_hbm, py_hbm,
               g1_hbm, g2_hbm, d2_hbm,
               pxv, pyv, *slot_refs):
    w = lax.axis_index("s") * NC + lax.axis_index("c")
    pltpu.sync_copy(px_hbm, pxv)
    pltpu.sync_copy(py_hbm, pyv)
    idxs = slot_refs[0:GSLOT]
    idxd = slot_refs[GSLOT:2 * GSLOT]
    b1 = slot_refs[2 * GSLOT:3 * GSLOT]
    b2 = slot_refs[3 * GSLOT:4 * GSLOT]
    d2b = slot_refs[4 * GSLOT:5 * GSLOT]
    sg = slot_refs[5 * GSLOT:6 * GSLOT]
    sw = slot_refs[6 * GSLOT:7 * GSLOT]

    def base_of(k):
        return (k * NW + w) * EC

    def load_idx(k, b):
        base = base_of(k)
        pltpu.sync_copy(src_hbm.at[pl.ds(base, EC)], idxs[b])
        pltpu.sync_copy(dst_hbm.at[pl.ds(base, EC)], idxd[b])

    def gather_copies(b):
        return (pltpu.make_async_copy(t1_hbm.at[idxs[b]], b1[b], sg[b]),
                pltpu.make_async_copy(t2_hbm.at[idxd[b]], b2[b], sg[b]))

    def wb_copies(k, b):
        base = base_of(k)
        return (pltpu.make_async_copy(b1[b], g1_hbm.at[pl.ds(base, EC)], sw[b]),
                pltpu.make_async_copy(b2[b], g2_hbm.at[pl.ds(base, EC)], sw[b]),
                pltpu.make_async_copy(d2b[b], d2_hbm.at[pl.ds(base, EC)], sw[b]))

    def compute_d2(b):
        @pl.loop(0, EC, step=16)
        def _(i):
            i16s = idxs[b][pl.ds(i, 16)]
            i16d = idxd[b][pl.ds(i, 16)]
            dx = plsc.load_gather(pxv, [i16s]) - plsc.load_gather(pxv, [i16d])
            dy = plsc.load_gather(pyv, [i16s]) - plsc.load_gather(pyv, [i16d])
            d2b[b][pl.ds(i, 16)] = dx * dx + dy * dy

    for b in range(GSLOT):
        load_idx(b, b)
        for cp in gather_copies(b):
            cp.start()

    @pl.loop(0, CPW, step=GSLOT)
    def _(k0):
        for b in range(GSLOT):
            k = k0 + b

            @pl.when(k * NW + w < NCHUNK)
            def _(k=k, b=b):
                for cp in gather_copies(b):
                    cp.wait()
                compute_d2(b)
                for cp in wb_copies(k, b):
                    cp.start()

                @pl.when((k + GSLOT) * NW + w < NCHUNK)
                def _():
                    load_idx(k + GSLOT, b)
                    for cp in wb_copies(k, b):
                        cp.wait()
                    for cp in gather_copies(b):
                        cp.start()

                @pl.when((k + GSLOT) * NW + w >= NCHUNK)
                def _():
                    for cp in wb_copies(k, b):
                        cp.wait()


def _scatter_sc(msg_hbm, src_hbm, mi_hbm, zbuf, acc, *slot_refs):
    c = lax.axis_index("c")
    s = lax.axis_index("s")
    w = s * NC + c
    idx = slot_refs[0:SSLOT]
    mb = slot_refs[SSLOT:2 * SSLOT]
    sl = slot_refs[2 * SSLOT:3 * SSLOT]

    z16 = jnp.zeros((16,), _f32)

    @pl.loop(0, ZR)
    def _(r):
        @pl.loop(0, H, step=16)
        def _(col):
            zbuf[r, pl.ds(col, 16)] = z16

    @pl.loop(0, RPS // ZR)
    def _(i):
        pltpu.sync_copy(zbuf, acc.at[pl.ds(s * RPS + i * ZR, ZR)])

    plsc.subcore_barrier()

    def base_of(k):
        return (k * NW + w) * EC

    def load_copies(k, b):
        base = base_of(k)
        return (pltpu.make_async_copy(src_hbm.at[pl.ds(base, EC)], idx[b], sl[b]),
                pltpu.make_async_copy(msg_hbm.at[pl.ds(base, EC)], mb[b], sl[b]))

    for b in range(SSLOT):
        for cp in load_copies(b, b):
            cp.start()

    @pl.loop(0, CPW, step=SSLOT)
    def _(k0):
        for b in range(SSLOT):
            k = k0 + b

            @pl.when(k * NW + w < NCHUNK)
            def _(k=k, b=b):
                for cp in load_copies(k, b):
                    cp.wait()
                pltpu.sync_copy(mb[b], acc.at[idx[b]], add=True)

                @pl.when((k + SSLOT) * NW + w < NCHUNK)
                def _():
                    for cp in load_copies(k + SSLOT, b):
                        cp.start()

    plsc.subcore_barrier()
    pltpu.sync_copy(acc.at[pl.ds(s * RPS, RPS)], mi_hbm.at[c, pl.ds(s * RPS, RPS)])


def _full(shape):
    return pl.BlockSpec(shape, lambda i: tuple(0 for _ in shape))


def kernel(x, edge_index, W_emb, b_emb, W1, b1, W2, b2, Wi, bi,
           Wh1, bh1, Wh2, bh2, Wl1, bl1, Wl2, bl2):
    x = x.astype(_f32)
    src = edge_index[0].astype(jnp.int32)
    dst = edge_index[1].astype(jnp.int32)
    px = x[:, 0]
    py = x[:, 1]
    _sc_mesh = plsc.VectorSubcoreMesh(core_axis_name="c", subcore_axis_name="s")
    _sc_cp = pltpu.CompilerParams()
    if "needs_layout_passes" in pltpu.CompilerParams.__dataclass_fields__:
        _sc_cp = dataclasses.replace(_sc_cp, needs_layout_passes=False)

    h, t1, t2 = pl.pallas_call(
        _prep_body,
        grid=(N // RB,),
        in_specs=[
            pl.BlockSpec((RB, H), lambda i: (i, 0)),
            _full((H, H)), _full((1, H)), _full((H, H)), _full((H, H)), _full((1, H)),
        ],
        out_specs=[
            pl.BlockSpec((RB, H), lambda i: (i, 0)),
            pl.BlockSpec((RB, H), lambda i: (i, 0)),
            pl.BlockSpec((RB, H), lambda i: (i, 0)),
        ],
        out_shape=[
            jax.ShapeDtypeStruct((N, H), _f32),
            jax.ShapeDtypeStruct((N, H), _f32),
            jax.ShapeDtypeStruct((N, H), _f32),
        ],
    )(x, W_emb, b_emb.reshape(1, H), W1[:H], W1[H:2 * H], b1.reshape(1, H))

    gather = pl.kernel(
        _gather_sc,
        out_type=[
            jax.ShapeDtypeStruct((ES, H), _f32),
            jax.ShapeDtypeStruct((ES, H), _f32),
            jax.ShapeDtypeStruct((ES,), _f32),
        ],
        mesh=_sc_mesh,
        scratch_types=(
            [pltpu.VMEM((N,), _f32), pltpu.VMEM((N,), _f32)]
            + [pltpu.VMEM((EC,), jnp.int32)] * GSLOT
            + [pltpu.VMEM((EC,), jnp.int32)] * GSLOT
            + [pltpu.VMEM((EC, H), _f32)] * GSLOT
            + [pltpu.VMEM((EC, H), _f32)] * GSLOT
            + [pltpu.VMEM((EC,), _f32)] * GSLOT
            + [pltpu.SemaphoreType.DMA] * (2 * GSLOT)
        ),
        compiler_params=_sc_cp,
    )

    scatter = pl.kernel(
        _scatter_sc,
        out_type=jax.ShapeDtypeStruct((NC, NPAD, H), _f32),
        mesh=_sc_mesh,
        scratch_types=(
            [pltpu.VMEM((ZR, H), _f32), pltpu.VMEM_SHARED((NPAD, H), _f32)]
            + [pltpu.VMEM((EC,), jnp.int32)] * SSLOT
            + [pltpu.VMEM((EC, H), _f32)] * SSLOT
            + [pltpu.SemaphoreType.DMA] * SSLOT
        ),
    )

    w1dmat = jnp.tile(W1[2 * H].reshape(1, H), (H, 1))
    wimat = jnp.tile(Wi.reshape(H, 1), (1, H))
    mi_parts = []
    for sidx in range(SLABS):
        src_s = lax.slice_in_dim(src, sidx * ES, (sidx + 1) * ES)
        dst_s = lax.slice_in_dim(dst, sidx * ES, (sidx + 1) * ES)
        g1, g2, d2 = gather(t1, t2, src_s, dst_s, px, py)
        msg = pl.pallas_call(
            _edge_body,
            grid=(ES // EB,),
            in_specs=[
                pl.BlockSpec((EB, H), lambda i: (i, 0)),
                pl.BlockSpec((EB, H), lambda i: (i, 0)),
                pl.BlockSpec((1, EB // H, H), lambda i: (i, 0, 0)),
                _full((H, H)), _full((H, H)), _full((1, H)), _full((H, H)), _full((1, 1)),
            ],
            out_specs=pl.BlockSpec((EB, H), lambda i: (i, 0)),
            out_shape=jax.ShapeDtypeStruct((ES, H), _f32),
        )(g1, g2, d2.reshape(ES // EB, EB // H, H),
          w1dmat, W2, b2.reshape(1, H), wimat, bi.reshape(1, 1))
        mi_parts.append(scatter(msg, src_s))

    out = pl.pallas_call(
        _node_body,
        grid=(N // RB,),
        in_specs=(
            [pl.BlockSpec((RB, H), lambda i: (i, 0))]
            + [pl.BlockSpec((NC, RB, H), lambda i: (0, i, 0))] * SLABS
            + [_full((H, H)), _full((H, H)), _full((1, H)),
               _full((H, H)), _full((1, H)),
               _full((H, H)), _full((1, H)), _full((1, H)), _full((1, 1))]
        ),
        out_specs=pl.BlockSpec((RB, 1), lambda i: (i, 0)),
        out_shape=jax.ShapeDtypeStruct((N, 1), _f32),
    )(h, *mi_parts, Wh1[:H], Wh1[H:], bh1.reshape(1, H),
      Wh2, bh2.reshape(1, H), Wl1, bl1.reshape(1, H), Wl2.reshape(1, H),
      bl2.reshape(1, 1))

    return out


# speedup vs baseline: 1.0862x; 1.0519x over previous
import dataclasses

import jax
import jax.numpy as jnp
from jax import lax
from jax.experimental import pallas as pl
from jax.experimental.pallas import tpu as pltpu
from jax.experimental.pallas import tpu_sc as plsc

N = 10000
E = 320000
H = 128
NC = 2
NS = 16
NW = NC * NS
EC = 80
SLABS = 1
ES = E // SLABS
NCHUNK = ES // EC
CPW = -(-NCHUNK // NW)
NPAD = 10240
RPS = NPAD // NS
ZR = 64

RB = 2000
EB = 6400

_f32 = jnp.float32


def _prep_body(x_ref, wemb_ref, bemb_ref, w1a_ref, w1b_ref, b1_ref,
               h_ref, t1_ref, t2_ref):
    xb = x_ref[...]
    h = jnp.dot(xb, wemb_ref[...], preferred_element_type=_f32) + bemb_ref[...]
    h_ref[...] = h
    t1_ref[...] = jnp.dot(h, w1a_ref[...], preferred_element_type=_f32)
    t2_ref[...] = jnp.dot(h, w1b_ref[...], preferred_element_type=_f32) + b1_ref[...]


def _edge_body(g1_ref, g2_ref, d2_ref, w1dmat_ref, w2_ref, b2_ref, wimat_ref,
               bi_ref, msg_ref):
    s = jnp.sqrt(d2_ref[0])
    r_i = jax.lax.broadcasted_iota(jnp.int32, (EB, EB // H), 0)
    c_i = jax.lax.broadcasted_iota(jnp.int32, (EB, EB // H), 1)
    rep = (c_i == r_i // H).astype(_f32)
    drows = jnp.dot(rep, s, preferred_element_type=_f32)
    r2 = jax.lax.broadcasted_iota(jnp.int32, (EB, H), 0)
    j2 = jax.lax.broadcasted_iota(jnp.int32, (EB, H), 1)
    dsel = jnp.where(j2 == r2 % H, drows, 0.0)
    distw = jnp.dot(dsel, w1dmat_ref[...], preferred_element_type=_f32)
    z1 = g1_ref[...] + g2_ref[...] + distw
    z1b = z1.astype(jnp.bfloat16)
    u = z1b * jax.nn.sigmoid(z1b)
    v0 = (jnp.dot(u, w2_ref[...].astype(jnp.bfloat16),
                  preferred_element_type=_f32) + b2_ref[...])
    v0b = v0.astype(jnp.bfloat16)
    v = v0b * jax.nn.sigmoid(v0b)
    p = (jnp.dot(v, wimat_ref[...].astype(jnp.bfloat16),
                 preferred_element_type=_f32) + bi_ref[...]).astype(jnp.bfloat16)
    gate = jax.nn.sigmoid(p)
    msg_ref[...] = (gate * v).astype(_f32)


def _node_body(h_ref, *refs):
    mi_refs = refs[:SLABS]
    (wh1a_ref, wh1b_ref, bh1_ref, wh2_ref, bh2_ref,
     wl1_ref, bl1_ref, wl2_ref, bl2_ref, out_ref) = refs[SLABS:]
    h = h_ref[...]
    m = mi_refs[0][0] + mi_refs[0][1]
    for r in mi_refs[1:]:
        m = m + r[0] + r[1]
    t0 = (jnp.dot(h, wh1a_ref[...], preferred_element_type=_f32)
          + jnp.dot(m, wh1b_ref[...], preferred_element_type=_f32)
          + bh1_ref[...])
    t = t0 * jax.nn.sigmoid(t0)
    h2 = h + jnp.dot(t, wh2_ref[...], preferred_element_type=_f32) + bh2_ref[...]
    z = jnp.tanh(jnp.dot(h2, wl1_ref[...], preferred_element_type=_f32) + bl1_ref[...])
    out_ref[...] = jnp.sum(z * wl2_ref[...], axis=1, keepdims=True) + bl2_ref[...]


GSLOT = 5
SSLOT = 3


def _gather_sc(t1_hbm, t2_hbm, src_hbm, dst_hbm, px_hbm, py_hbm,
               g1_hbm, g2_hbm, d2_hbm,
               pxv, pyv, *slot_refs):
    w = lax.axis_index("s") * NC + lax.axis_index("c")
    pltpu.sync_copy(px_hbm, pxv)
    pltpu.sync_copy(py_hbm, pyv)
    idxs = slot_refs[0:GSLOT]
    idxd = slot_refs[GSLOT:2 * GSLOT]
    b1 = slot_refs[2 * GSLOT:3 * GSLOT]
    b2 = slot_refs[3 * GSLOT:4 * GSLOT]
    d2b = slot_refs[4 * GSLOT:5 * GSLOT]
    sg = slot_refs[5 * GSLOT:6 * GSLOT]
    sw = slot_refs[6 * GSLOT:7 * GSLOT]

    def base_of(k):
        return (k * NW + w) * EC

    def load_idx(k, b):
        base = base_of(k)
        pltpu.sync_copy(src_hbm.at[pl.ds(base, EC)], idxs[b])
        pltpu.sync_copy(dst_hbm.at[pl.ds(base, EC)], idxd[b])

    def gather_copies(b):
        return (pltpu.make_async_copy(t1_hbm.at[idxs[b]], b1[b], sg[b]),
                pltpu.make_async_copy(t2_hbm.at[idxd[b]], b2[b], sg[b]))

    def wb_copies(k, b):
        base = base_of(k)
        return (pltpu.make_async_copy(b1[b], g1_hbm.at[pl.ds(base, EC)], sw[b]),
                pltpu.make_async_copy(b2[b], g2_hbm.at[pl.ds(base, EC)], sw[b]),
                pltpu.make_async_copy(d2b[b], d2_hbm.at[pl.ds(base, EC)], sw[b]))

    def compute_d2(b):
        @pl.loop(0, EC, step=16)
        def _(i):
            i16s = idxs[b][pl.ds(i, 16)]
            i16d = idxd[b][pl.ds(i, 16)]
            dx = plsc.load_gather(pxv, [i16s]) - plsc.load_gather(pxv, [i16d])
            dy = plsc.load_gather(pyv, [i16s]) - plsc.load_gather(pyv, [i16d])
            d2b[b][pl.ds(i, 16)] = dx * dx + dy * dy

    for b in range(GSLOT):
        load_idx(b, b)
        for cp in gather_copies(b):
            cp.start()

    @pl.loop(0, CPW, step=GSLOT)
    def _(k0):
        for b in range(GSLOT):
            k = k0 + b

            @pl.when(k * NW + w < NCHUNK)
            def _(k=k, b=b):
                for cp in gather_copies(b):
                    cp.wait()
                compute_d2(b)
                for cp in wb_copies(k, b):
                    cp.start()

                @pl.when((k + GSLOT) * NW + w < NCHUNK)
                def _():
                    load_idx(k + GSLOT, b)
                    for cp in wb_copies(k, b):
                        cp.wait()
                    for cp in gather_copies(b):
                        cp.start()

                @pl.when((k + GSLOT) * NW + w >= NCHUNK)
                def _():
                    for cp in wb_copies(k, b):
                        cp.wait()


def _scatter_sc(msg_hbm, src_hbm, mi_hbm, zbuf, acc, *slot_refs):
    c = lax.axis_index("c")
    s = lax.axis_index("s")
    w = s * NC + c
    idx = slot_refs[0:SSLOT]
    mb = slot_refs[SSLOT:2 * SSLOT]
    sl = slot_refs[2 * SSLOT:3 * SSLOT]

    z16 = jnp.zeros((16,), _f32)

    @pl.loop(0, ZR)
    def _(r):
        @pl.loop(0, H, step=16)
        def _(col):
            zbuf[r, pl.ds(col, 16)] = z16

    @pl.loop(0, RPS // ZR)
    def _(i):
        pltpu.sync_copy(zbuf, acc.at[pl.ds(s * RPS + i * ZR, ZR)])

    plsc.subcore_barrier()

    def base_of(k):
        return (k * NW + w) * EC

    def load_copies(k, b):
        base = base_of(k)
        return (pltpu.make_async_copy(src_hbm.at[pl.ds(base, EC)], idx[b], sl[b]),
                pltpu.make_async_copy(msg_hbm.at[pl.ds(base, EC)], mb[b], sl[b]))

    for b in range(SSLOT):
        for cp in load_copies(b, b):
            cp.start()

    @pl.loop(0, CPW, step=SSLOT)
    def _(k0):
        for b in range(SSLOT):
            k = k0 + b

            @pl.when(k * NW + w < NCHUNK)
            def _(k=k, b=b):
                for cp in load_copies(k, b):
                    cp.wait()
                pltpu.sync_copy(mb[b], acc.at[idx[b]], add=True)

                @pl.when((k + SSLOT) * NW + w < NCHUNK)
                def _():
                    for cp in load_copies(k + SSLOT, b):
                        cp.start()

    plsc.subcore_barrier()
    pltpu.sync_copy(acc.at[pl.ds(s * RPS, RPS)], mi_hbm.at[c, pl.ds(s * RPS, RPS)])


def _full(shape):
    return pl.BlockSpec(shape, lambda i: tuple(0 for _ in shape))


def kernel(x, edge_index, W_emb, b_emb, W1, b1, W2, b2, Wi, bi,
           Wh1, bh1, Wh2, bh2, Wl1, bl1, Wl2, bl2):
    x = x.astype(_f32)
    src = edge_index[0].astype(jnp.int32)
    dst = edge_index[1].astype(jnp.int32)
    px = x[:, 0]
    py = x[:, 1]
    _sc_mesh = plsc.VectorSubcoreMesh(core_axis_name="c", subcore_axis_name="s")
    _sc_cp = pltpu.CompilerParams()
    if "needs_layout_passes" in pltpu.CompilerParams.__dataclass_fields__:
        _sc_cp = dataclasses.replace(_sc_cp, needs_layout_passes=False)

    h, t1, t2 = pl.pallas_call(
        _prep_body,
        grid=(N // RB,),
        in_specs=[
            pl.BlockSpec((RB, H), lambda i: (i, 0)),
            _full((H, H)), _full((1, H)), _full((H, H)), _full((H, H)), _full((1, H)),
        ],
        out_specs=[
            pl.BlockSpec((RB, H), lambda i: (i, 0)),
            pl.BlockSpec((RB, H), lambda i: (i, 0)),
            pl.BlockSpec((RB, H), lambda i: (i, 0)),
        ],
        out_shape=[
            jax.ShapeDtypeStruct((N, H), _f32),
            jax.ShapeDtypeStruct((N, H), _f32),
            jax.ShapeDtypeStruct((N, H), _f32),
        ],
    )(x, W_emb, b_emb.reshape(1, H), W1[:H], W1[H:2 * H], b1.reshape(1, H))

    gather = pl.kernel(
        _gather_sc,
        out_type=[
            jax.ShapeDtypeStruct((ES, H), _f32),
            jax.ShapeDtypeStruct((ES, H), _f32),
            jax.ShapeDtypeStruct((ES,), _f32),
        ],
        mesh=_sc_mesh,
        scratch_types=(
            [pltpu.VMEM((N,), _f32), pltpu.VMEM((N,), _f32)]
            + [pltpu.VMEM((EC,), jnp.int32)] * GSLOT
            + [pltpu.VMEM((EC,), jnp.int32)] * GSLOT
            + [pltpu.VMEM((EC, H), _f32)] * GSLOT
            + [pltpu.VMEM((EC, H), _f32)] * GSLOT
            + [pltpu.VMEM((EC,), _f32)] * GSLOT
            + [pltpu.SemaphoreType.DMA] * (2 * GSLOT)
        ),
        compiler_params=_sc_cp,
    )

    scatter = pl.kernel(
        _scatter_sc,
        out_type=jax.ShapeDtypeStruct((NC, NPAD, H), _f32),
        mesh=_sc_mesh,
        scratch_types=(
            [pltpu.VMEM((ZR, H), _f32), pltpu.VMEM_SHARED((NPAD, H), _f32)]
            + [pltpu.VMEM((EC,), jnp.int32)] * SSLOT
            + [pltpu.VMEM((EC, H), _f32)] * SSLOT
            + [pltpu.SemaphoreType.DMA] * SSLOT
        ),
    )

    w1dmat = jnp.tile(W1[2 * H].reshape(1, H), (H, 1))
    wimat = jnp.tile(Wi.reshape(H, 1), (1, H))
    mi_parts = []
    for sidx in range(SLABS):
        src_s = lax.slice_in_dim(src, sidx * ES, (sidx + 1) * ES)
        dst_s = lax.slice_in_dim(dst, sidx * ES, (sidx + 1) * ES)
        g1, g2, d2 = gather(t1, t2, src_s, dst_s, px, py)
        msg = pl.pallas_call(
            _edge_body,
            grid=(ES // EB,),
            in_specs=[
                pl.BlockSpec((EB, H), lambda i: (i, 0)),
                pl.BlockSpec((EB, H), lambda i: (i, 0)),
                pl.BlockSpec((1, EB // H, H), lambda i: (i, 0, 0)),
                _full((H, H)), _full((H, H)), _full((1, H)), _full((H, H)), _full((1, 1)),
            ],
            out_specs=pl.BlockSpec((EB, H), lambda i: (i, 0)),
            out_shape=jax.ShapeDtypeStruct((ES, H), _f32),
        )(g1, g2, d2.reshape(ES // EB, EB // H, H),
          w1dmat, W2, b2.reshape(1, H), wimat, bi.reshape(1, 1))
        mi_parts.append(scatter(msg, src_s))

    out = pl.pallas_call(
        _node_body,
        grid=(N // RB,),
        in_specs=(
            [pl.BlockSpec((RB, H), lambda i: (i, 0))]
            + [pl.BlockSpec((NC, RB, H), lambda i: (0, i, 0))] * SLABS
            + [_full((H, H)), _full((H, H)), _full((1, H)),
               _full((H, H)), _full((1, H)),
               _full((H, H)), _full((1, H)), _full((1, H)), _full((1, 1))]
        ),
        out_specs=pl.BlockSpec((RB, 1), lambda i: (i, 0)),
        out_shape=jax.ShapeDtypeStruct((N, 1), _f32),
    )(h, *mi_parts, Wh1[:H], Wh1[H:], bh1.reshape(1, H),
      Wh2, bh2.reshape(1, H), Wl1, bl1.reshape(1, H), Wl2.reshape(1, H),
      bl2.reshape(1, 1))

    return out


# speedup vs baseline: 1.0865x; 1.0002x over previous
import dataclasses

import jax
import jax.numpy as jnp
from jax import lax
from jax.experimental import pallas as pl
from jax.experimental.pallas import tpu as pltpu
from jax.experimental.pallas import tpu_sc as plsc

N = 10000
E = 320000
H = 128
NC = 2
NS = 16
NW = NC * NS
EC = 80
SLABS = 1
ES = E // SLABS
NCHUNK = ES // EC
CPW = -(-NCHUNK // NW)
NPAD = 10240
RPS = NPAD // NS
ZR = 32

RB = 2000
EB = 6400

_f32 = jnp.float32


def _prep_body(x_ref, wemb_ref, bemb_ref, w1a_ref, w1b_ref, b1_ref,
               h_ref, t1_ref, t2_ref):
    xb = x_ref[...]
    h = jnp.dot(xb, wemb_ref[...], preferred_element_type=_f32) + bemb_ref[...]
    h_ref[...] = h
    t1_ref[...] = jnp.dot(h, w1a_ref[...], preferred_element_type=_f32)
    t2_ref[...] = jnp.dot(h, w1b_ref[...], preferred_element_type=_f32) + b1_ref[...]


def _edge_body(g1_ref, g2_ref, d2_ref, w1dmat_ref, w2_ref, b2_ref, wimat_ref,
               bi_ref, msg_ref):
    s = jnp.sqrt(d2_ref[0])
    r_i = jax.lax.broadcasted_iota(jnp.int32, (EB, EB // H), 0)
    c_i = jax.lax.broadcasted_iota(jnp.int32, (EB, EB // H), 1)
    rep = (c_i == r_i // H).astype(_f32)
    drows = jnp.dot(rep, s, preferred_element_type=_f32)
    r2 = jax.lax.broadcasted_iota(jnp.int32, (EB, H), 0)
    j2 = jax.lax.broadcasted_iota(jnp.int32, (EB, H), 1)
    dsel = jnp.where(j2 == r2 % H, drows, 0.0)
    distw = jnp.dot(dsel, w1dmat_ref[...], preferred_element_type=_f32)
    z1 = g1_ref[...] + g2_ref[...] + distw
    z1b = z1.astype(jnp.bfloat16)
    u = z1b * jax.nn.sigmoid(z1b)
    v0 = (jnp.dot(u, w2_ref[...].astype(jnp.bfloat16),
                  preferred_element_type=_f32) + b2_ref[...])
    v0b = v0.astype(jnp.bfloat16)
    v = v0b * jax.nn.sigmoid(v0b)
    p = (jnp.dot(v, wimat_ref[...].astype(jnp.bfloat16),
                 preferred_element_type=_f32) + bi_ref[...]).astype(jnp.bfloat16)
    gate = jax.nn.sigmoid(p)
    msg_ref[...] = (gate * v).astype(_f32)


def _node_body(h_ref, *refs):
    mi_refs = refs[:SLABS]
    (wh1a_ref, wh1b_ref, bh1_ref, wh2_ref, bh2_ref,
     wl1_ref, bl1_ref, wl2_ref, bl2_ref, out_ref) = refs[SLABS:]
    h = h_ref[...]
    m = mi_refs[0][0] + mi_refs[0][1]
    for r in mi_refs[1:]:
        m = m + r[0] + r[1]
    t0 = (jnp.dot(h, wh1a_ref[...], preferred_element_type=_f32)
          + jnp.dot(m, wh1b_ref[...], preferred_element_type=_f32)
          + bh1_ref[...])
    t = t0 * jax.nn.sigmoid(t0)
    h2 = h + jnp.dot(t, wh2_ref[...], preferred_element_type=_f32) + bh2_ref[...]
    z = jnp.tanh(jnp.dot(h2, wl1_ref[...], preferred_element_type=_f32) + bl1_ref[...])
    out_ref[...] = jnp.sum(z * wl2_ref[...], axis=1, keepdims=True) + bl2_ref[...]


GSLOT = 5
SSLOT = 4


def _gather_sc(t1_hbm, t2_hbm, src_hbm, dst_hbm, px_hbm, py_hbm,
               g1_hbm, g2_hbm, d2_hbm,
               pxv, pyv, *slot_refs):
    w = lax.axis_index("s") * NC + lax.axis_index("c")
    pltpu.sync_copy(px_hbm, pxv)
    pltpu.sync_copy(py_hbm, pyv)
    idxs = slot_refs[0:GSLOT]
    idxd = slot_refs[GSLOT:2 * GSLOT]
    b1 = slot_refs[2 * GSLOT:3 * GSLOT]
    b2 = slot_refs[3 * GSLOT:4 * GSLOT]
    d2b = slot_refs[4 * GSLOT:5 * GSLOT]
    sg = slot_refs[5 * GSLOT:6 * GSLOT]
    sw = slot_refs[6 * GSLOT:7 * GSLOT]

    def base_of(k):
        return (k * NW + w) * EC

    def load_idx(k, b):
        base = base_of(k)
        pltpu.sync_copy(src_hbm.at[pl.ds(base, EC)], idxs[b])
        pltpu.sync_copy(dst_hbm.at[pl.ds(base, EC)], idxd[b])

    def gather_copies(b):
        return (pltpu.make_async_copy(t1_hbm.at[idxs[b]], b1[b], sg[b]),
                pltpu.make_async_copy(t2_hbm.at[idxd[b]], b2[b], sg[b]))

    def wb_copies(k, b):
        base = base_of(k)
        return (pltpu.make_async_copy(b1[b], g1_hbm.at[pl.ds(base, EC)], sw[b]),
                pltpu.make_async_copy(b2[b], g2_hbm.at[pl.ds(base, EC)], sw[b]),
                pltpu.make_async_copy(d2b[b], d2_hbm.at[pl.ds(base, EC)], sw[b]))

    def compute_d2(b):
        @pl.loop(0, EC, step=16)
        def _(i):
            i16s = idxs[b][pl.ds(i, 16)]
            i16d = idxd[b][pl.ds(i, 16)]
            dx = plsc.load_gather(pxv, [i16s]) - plsc.load_gather(pxv, [i16d])
            dy = plsc.load_gather(pyv, [i16s]) - plsc.load_gather(pyv, [i16d])
            d2b[b][pl.ds(i, 16)] = dx * dx + dy * dy

    for b in range(GSLOT):
        load_idx(b, b)
        for cp in gather_copies(b):
            cp.start()

    @pl.loop(0, CPW, step=GSLOT)
    def _(k0):
        for b in range(GSLOT):
            k = k0 + b

            @pl.when(k * NW + w < NCHUNK)
            def _(k=k, b=b):
                for cp in gather_copies(b):
                    cp.wait()
                compute_d2(b)
                for cp in wb_copies(k, b):
                    cp.start()

                @pl.when((k + GSLOT) * NW + w < NCHUNK)
                def _():
                    load_idx(k + GSLOT, b)
                    for cp in wb_copies(k, b):
                        cp.wait()
                    for cp in gather_copies(b):
                        cp.start()

                @pl.when((k + GSLOT) * NW + w >= NCHUNK)
                def _():
                    for cp in wb_copies(k, b):
                        cp.wait()


def _scatter_sc(msg_hbm, src_hbm, mi_hbm, zbuf, acc, *slot_refs):
    c = lax.axis_index("c")
    s = lax.axis_index("s")
    w = s * NC + c
    idx = slot_refs[0:SSLOT]
    mb = slot_refs[SSLOT:2 * SSLOT]
    sl = slot_refs[2 * SSLOT:3 * SSLOT]

    z16 = jnp.zeros((16,), _f32)

    @pl.loop(0, ZR)
    def _(r):
        @pl.loop(0, H, step=16)
        def _(col):
            zbuf[r, pl.ds(col, 16)] = z16

    @pl.loop(0, RPS // ZR)
    def _(i):
        pltpu.sync_copy(zbuf, acc.at[pl.ds(s * RPS + i * ZR, ZR)])

    plsc.subcore_barrier()

    def base_of(k):
        return (k * NW + w) * EC

    def load_copies(k, b):
        base = base_of(k)
        return (pltpu.make_async_copy(src_hbm.at[pl.ds(base, EC)], idx[b], sl[b]),
                pltpu.make_async_copy(msg_hbm.at[pl.ds(base, EC)], mb[b], sl[b]))

    for b in range(SSLOT):
        for cp in load_copies(b, b):
            cp.start()

    @pl.loop(0, CPW, step=SSLOT)
    def _(k0):
        for b in range(SSLOT):
            k = k0 + b

            @pl.when(k * NW + w < NCHUNK)
            def _(k=k, b=b):
                for cp in load_copies(k, b):
                    cp.wait()
                pltpu.sync_copy(mb[b], acc.at[idx[b]], add=True)

                @pl.when((k + SSLOT) * NW + w < NCHUNK)
                def _():
                    for cp in load_copies(k + SSLOT, b):
                        cp.start()

    plsc.subcore_barrier()
    pltpu.sync_copy(acc.at[pl.ds(s * RPS, RPS)], mi_hbm.at[c, pl.ds(s * RPS, RPS)])


def _full(shape):
    return pl.BlockSpec(shape, lambda i: tuple(0 for _ in shape))


def kernel(x, edge_index, W_emb, b_emb, W1, b1, W2, b2, Wi, bi,
           Wh1, bh1, Wh2, bh2, Wl1, bl1, Wl2, bl2):
    x = x.astype(_f32)
    src = edge_index[0].astype(jnp.int32)
    dst = edge_index[1].astype(jnp.int32)
    px = x[:, 0]
    py = x[:, 1]
    _sc_mesh = plsc.VectorSubcoreMesh(core_axis_name="c", subcore_axis_name="s")
    _sc_cp = pltpu.CompilerParams()
    if "needs_layout_passes" in pltpu.CompilerParams.__dataclass_fields__:
        _sc_cp = dataclasses.replace(_sc_cp, needs_layout_passes=False)

    h, t1, t2 = pl.pallas_call(
        _prep_body,
        grid=(N // RB,),
        in_specs=[
            pl.BlockSpec((RB, H), lambda i: (i, 0)),
            _full((H, H)), _full((1, H)), _full((H, H)), _full((H, H)), _full((1, H)),
        ],
        out_specs=[
            pl.BlockSpec((RB, H), lambda i: (i, 0)),
            pl.BlockSpec((RB, H), lambda i: (i, 0)),
            pl.BlockSpec((RB, H), lambda i: (i, 0)),
        ],
        out_shape=[
            jax.ShapeDtypeStruct((N, H), _f32),
            jax.ShapeDtypeStruct((N, H), _f32),
            jax.ShapeDtypeStruct((N, H), _f32),
        ],
    )(x, W_emb, b_emb.reshape(1, H), W1[:H], W1[H:2 * H], b1.reshape(1, H))

    gather = pl.kernel(
        _gather_sc,
        out_type=[
            jax.ShapeDtypeStruct((ES, H), _f32),
            jax.ShapeDtypeStruct((ES, H), _f32),
            jax.ShapeDtypeStruct((ES,), _f32),
        ],
        mesh=_sc_mesh,
        scratch_types=(
            [pltpu.VMEM((N,), _f32), pltpu.VMEM((N,), _f32)]
            + [pltpu.VMEM((EC,), jnp.int32)] * GSLOT
            + [pltpu.VMEM((EC,), jnp.int32)] * GSLOT
            + [pltpu.VMEM((EC, H), _f32)] * GSLOT
            + [pltpu.VMEM((EC, H), _f32)] * GSLOT
            + [pltpu.VMEM((EC,), _f32)] * GSLOT
            + [pltpu.SemaphoreType.DMA] * (2 * GSLOT)
        ),
        compiler_params=_sc_cp,
    )

    scatter = pl.kernel(
        _scatter_sc,
        out_type=jax.ShapeDtypeStruct((NC, NPAD, H), _f32),
        mesh=_sc_mesh,
        scratch_types=(
            [pltpu.VMEM((ZR, H), _f32), pltpu.VMEM_SHARED((NPAD, H), _f32)]
            + [pltpu.VMEM((EC,), jnp.int32)] * SSLOT
            + [pltpu.VMEM((EC, H), _f32)] * SSLOT
            + [pltpu.SemaphoreType.DMA] * SSLOT
        ),
    )

    w1dmat = jnp.tile(W1[2 * H].reshape(1, H), (H, 1))
    wimat = jnp.tile(Wi.reshape(H, 1), (1, H))
    mi_parts = []
    for sidx in range(SLABS):
        src_s = lax.slice_in_dim(src, sidx * ES, (sidx + 1) * ES)
        dst_s = lax.slice_in_dim(dst, sidx * ES, (sidx + 1) * ES)
        g1, g2, d2 = gather(t1, t2, src_s, dst_s, px, py)
        msg = pl.pallas_call(
            _edge_body,
            grid=(ES // EB,),
            in_specs=[
                pl.BlockSpec((EB, H), lambda i: (i, 0)),
                pl.BlockSpec((EB, H), lambda i: (i, 0)),
                pl.BlockSpec((1, EB // H, H), lambda i: (i, 0, 0)),
                _full((H, H)), _full((H, H)), _full((1, H)), _full((H, H)), _full((1, 1)),
            ],
            out_specs=pl.BlockSpec((EB, H), lambda i: (i, 0)),
            out_shape=jax.ShapeDtypeStruct((ES, H), _f32),
        )(g1, g2, d2.reshape(ES // EB, EB // H, H),
          w1dmat, W2, b2.reshape(1, H), wimat, bi.reshape(1, 1))
        mi_parts.append(scatter(msg, src_s))

    out = pl.pallas_call(
        _node_body,
        grid=(N // RB,),
        in_specs=(
            [pl.BlockSpec((RB, H), lambda i: (i, 0))]
            + [pl.BlockSpec((NC, RB, H), lambda i: (0, i, 0))] * SLABS
            + [_full((H, H)), _full((H, H)), _full((1, H)),
               _full((H, H)), _full((1, H)),
               _full((H, H)), _full((1, H)), _full((1, H)), _full((1, 1))]
        ),
        out_specs=pl.BlockSpec((RB, 1), lambda i: (i, 0)),
        out_shape=jax.ShapeDtypeStruct((N, 1), _f32),
    )(h, *mi_parts, Wh1[:H], Wh1[H:], bh1.reshape(1, H),
      Wh2, bh2.reshape(1, H), Wl1, bl1.reshape(1, H), Wl2.reshape(1, H),
      bl2.reshape(1, 1))

    return out


# speedup vs baseline: 1.0961x; 1.0088x over previous
import dataclasses

import jax
import jax.numpy as jnp
from jax import lax
from jax.experimental import pallas as pl
from jax.experimental.pallas import tpu as pltpu
from jax.experimental.pallas import tpu_sc as plsc

N = 10000
E = 320000
H = 128
NC = 2
NS = 16
NW = NC * NS
EC = 80
SLABS = 1
ES = E // SLABS
NCHUNK = ES // EC
CPW = -(-NCHUNK // NW)
NPAD = 10240
RPS = NPAD // NS
ZR = 32

RB = 2000
EB = 6400

_f32 = jnp.float32


def _prep_body(x_ref, wemb_ref, bemb_ref, w1a_ref, w1b_ref, b1_ref,
               h_ref, t1_ref, t2_ref):
    xb = x_ref[...]
    h = jnp.dot(xb, wemb_ref[...], preferred_element_type=_f32) + bemb_ref[...]
    h_ref[...] = h
    t1_ref[...] = jnp.dot(h, w1a_ref[...], preferred_element_type=_f32)
    t2_ref[...] = jnp.dot(h, w1b_ref[...], preferred_element_type=_f32) + b1_ref[...]


def _edge_body(g_ref, d2_ref, w1dmat_ref, w2_ref, b2_ref, wimat_ref,
               bi_ref, msg_ref):
    s = jnp.sqrt(d2_ref[0])
    r_i = jax.lax.broadcasted_iota(jnp.int32, (EB, EB // H), 0)
    c_i = jax.lax.broadcasted_iota(jnp.int32, (EB, EB // H), 1)
    rep = (c_i == r_i // H).astype(_f32)
    drows = jnp.dot(rep, s, preferred_element_type=_f32)
    r2 = jax.lax.broadcasted_iota(jnp.int32, (EB, H), 0)
    j2 = jax.lax.broadcasted_iota(jnp.int32, (EB, H), 1)
    dsel = jnp.where(j2 == r2 % H, drows, 0.0)
    distw = jnp.dot(dsel, w1dmat_ref[...], preferred_element_type=_f32)
    z1 = g_ref[...] + distw
    z1b = z1.astype(jnp.bfloat16)
    u = z1b * jax.nn.sigmoid(z1b)
    v0 = (jnp.dot(u, w2_ref[...].astype(jnp.bfloat16),
                  preferred_element_type=_f32) + b2_ref[...])
    v0b = v0.astype(jnp.bfloat16)
    v = v0b * jax.nn.sigmoid(v0b)
    p = (jnp.dot(v, wimat_ref[...].astype(jnp.bfloat16),
                 preferred_element_type=_f32) + bi_ref[...]).astype(jnp.bfloat16)
    gate = jax.nn.sigmoid(p)
    msg_ref[...] = (gate * v).astype(_f32)


def _node_body(h_ref, *refs):
    mi_refs = refs[:SLABS]
    (wh1a_ref, wh1b_ref, bh1_ref, wh2_ref, bh2_ref,
     wl1_ref, bl1_ref, wl2_ref, bl2_ref, out_ref) = refs[SLABS:]
    h = h_ref[...]
    m = mi_refs[0][0] + mi_refs[0][1]
    for r in mi_refs[1:]:
        m = m + r[0] + r[1]
    t0 = (jnp.dot(h, wh1a_ref[...], preferred_element_type=_f32)
          + jnp.dot(m, wh1b_ref[...], preferred_element_type=_f32)
          + bh1_ref[...])
    t = t0 * jax.nn.sigmoid(t0)
    h2 = h + jnp.dot(t, wh2_ref[...], preferred_element_type=_f32) + bh2_ref[...]
    z = jnp.tanh(jnp.dot(h2, wl1_ref[...], preferred_element_type=_f32) + bl1_ref[...])
    out_ref[...] = jnp.sum(z * wl2_ref[...], axis=1, keepdims=True) + bl2_ref[...]


GSLOT = 4
SSLOT = 4


def _gather_sc(t1_hbm, t2_hbm, src_hbm, dst_hbm, px_hbm, py_hbm,
               g_hbm, d2_hbm,
               pxv, pyv, iv, Sv, *slot_refs):
    c = lax.axis_index("c")
    sid = lax.axis_index("s")
    w = sid * NC + c
    pltpu.sync_copy(px_hbm, pxv)
    pltpu.sync_copy(py_hbm, pyv)
    idxs = slot_refs[0:GSLOT]
    idxd = slot_refs[GSLOT:2 * GSLOT]
    b1 = slot_refs[2 * GSLOT:3 * GSLOT]
    b2 = slot_refs[3 * GSLOT:4 * GSLOT]
    d2b = slot_refs[4 * GSLOT:5 * GSLOT]
    sg = slot_refs[5 * GSLOT:6 * GSLOT]
    sw = slot_refs[6 * GSLOT:6 * GSLOT + 2]

    @pl.loop(0, EC, step=16)
    def _(i):
        iv[pl.ds(i, 16)] = lax.iota(jnp.int32, 16) + i

    def base_of(k):
        return (k * NW + w) * EC

    def load_idx(k, b):
        base = base_of(k)
        pltpu.sync_copy(src_hbm.at[pl.ds(base, EC)], idxs[b])
        pltpu.sync_copy(dst_hbm.at[pl.ds(base, EC)], idxd[b])

    def gather_copies(b):
        return (pltpu.make_async_copy(t1_hbm.at[idxs[b]], b1[b], sg[b]),
                pltpu.make_async_copy(t2_hbm.at[idxd[b]], b2[b], sg[b]))

    def wb_copy(k, ss):
        return pltpu.make_async_copy(
            Sv.at[sid, ss], g_hbm.at[pl.ds(base_of(k), EC)], sw[ss])

    def compute_d2(b):
        @pl.loop(0, EC, step=16)
        def _(i):
            i16s = idxs[b][pl.ds(i, 16)]
            i16d = idxd[b][pl.ds(i, 16)]
            dx = plsc.load_gather(pxv, [i16s]) - plsc.load_gather(pxv, [i16d])
            dy = plsc.load_gather(pyv, [i16s]) - plsc.load_gather(pyv, [i16d])
            d2b[b][pl.ds(i, 16)] = dx * dx + dy * dy

    for b in range(GSLOT):
        load_idx(b, b)
        for cp in gather_copies(b):
            cp.start()

    @pl.loop(0, CPW, step=GSLOT)
    def _(k0):
        for b in range(GSLOT):
            k = k0 + b
            ss = b % 2

            @pl.when(k < CPW)
            def _(k=k, b=b, ss=ss):
                for cp in gather_copies(b):
                    cp.wait()
                compute_d2(b)

                @pl.when(k >= 2)
                def _():
                    wb_copy(k - 2, ss).wait()

                pltpu.sync_copy(b1[b], Sv.at[sid, ss])
                pltpu.sync_copy(b2[b], Sv.at[sid, ss].at[iv], add=True)
                wb_copy(k, ss).start()
                pltpu.sync_copy(d2b[b], d2_hbm.at[pl.ds(base_of(k), EC)])

                @pl.when(k + GSLOT < CPW)
                def _():
                    load_idx(k + GSLOT, b)
                    for cp in gather_copies(b):
                        cp.start()

    for kk in (CPW - 2, CPW - 1):
        wb_copy(kk, kk % 2).wait()


def _scatter_sc(msg_hbm, src_hbm, mi_hbm, zbuf, acc, *slot_refs):
    c = lax.axis_index("c")
    s = lax.axis_index("s")
    w = s * NC + c
    idx = slot_refs[0:SSLOT]
    mb = slot_refs[SSLOT:2 * SSLOT]
    sl = slot_refs[2 * SSLOT:3 * SSLOT]

    z16 = jnp.zeros((16,), _f32)

    @pl.loop(0, ZR)
    def _(r):
        @pl.loop(0, H, step=16)
        def _(col):
            zbuf[r, pl.ds(col, 16)] = z16

    @pl.loop(0, RPS // ZR)
    def _(i):
        pltpu.sync_copy(zbuf, acc.at[pl.ds(s * RPS + i * ZR, ZR)])

    plsc.subcore_barrier()

    def base_of(k):
        return (k * NW + w) * EC

    def load_copies(k, b):
        base = base_of(k)
        return (pltpu.make_async_copy(src_hbm.at[pl.ds(base, EC)], idx[b], sl[b]),
                pltpu.make_async_copy(msg_hbm.at[pl.ds(base, EC)], mb[b], sl[b]))

    for b in range(SSLOT):
        for cp in load_copies(b, b):
            cp.start()

    @pl.loop(0, CPW, step=SSLOT)
    def _(k0):
        for b in range(SSLOT):
            k = k0 + b

            @pl.when(k * NW + w < NCHUNK)
            def _(k=k, b=b):
                for cp in load_copies(k, b):
                    cp.wait()
                pltpu.sync_copy(mb[b], acc.at[idx[b]], add=True)

                @pl.when((k + SSLOT) * NW + w < NCHUNK)
                def _():
                    for cp in load_copies(k + SSLOT, b):
                        cp.start()

    plsc.subcore_barrier()
    pltpu.sync_copy(acc.at[pl.ds(s * RPS, RPS)], mi_hbm.at[c, pl.ds(s * RPS, RPS)])


def _full(shape):
    return pl.BlockSpec(shape, lambda i: tuple(0 for _ in shape))


def kernel(x, edge_index, W_emb, b_emb, W1, b1, W2, b2, Wi, bi,
           Wh1, bh1, Wh2, bh2, Wl1, bl1, Wl2, bl2):
    x = x.astype(_f32)
    src = edge_index[0].astype(jnp.int32)
    dst = edge_index[1].astype(jnp.int32)
    px = x[:, 0]
    py = x[:, 1]
    _sc_mesh = plsc.VectorSubcoreMesh(core_axis_name="c", subcore_axis_name="s")
    _sc_cp = pltpu.CompilerParams()
    if "needs_layout_passes" in pltpu.CompilerParams.__dataclass_fields__:
        _sc_cp = dataclasses.replace(_sc_cp, needs_layout_passes=False)

    h, t1, t2 = pl.pallas_call(
        _prep_body,
        grid=(N // RB,),
        in_specs=[
            pl.BlockSpec((RB, H), lambda i: (i, 0)),
            _full((H, H)), _full((1, H)), _full((H, H)), _full((H, H)), _full((1, H)),
        ],
        out_specs=[
            pl.BlockSpec((RB, H), lambda i: (i, 0)),
            pl.BlockSpec((RB, H), lambda i: (i, 0)),
            pl.BlockSpec((RB, H), lambda i: (i, 0)),
        ],
        out_shape=[
            jax.ShapeDtypeStruct((N, H), _f32),
            jax.ShapeDtypeStruct((N, H), _f32),
            jax.ShapeDtypeStruct((N, H), _f32),
        ],
    )(x, W_emb, b_emb.reshape(1, H), W1[:H], W1[H:2 * H], b1.reshape(1, H))

    gather = pl.kernel(
        _gather_sc,
        out_type=[
            jax.ShapeDtypeStruct((ES, H), _f32),
            jax.ShapeDtypeStruct((ES,), _f32),
        ],
        mesh=_sc_mesh,
        scratch_types=(
            [pltpu.VMEM((N,), _f32), pltpu.VMEM((N,), _f32),
             pltpu.VMEM((EC,), jnp.int32),
             pltpu.VMEM_SHARED((NS, 2, EC, H), _f32)]
            + [pltpu.VMEM((EC,), jnp.int32)] * GSLOT
            + [pltpu.VMEM((EC,), jnp.int32)] * GSLOT
            + [pltpu.VMEM((EC, H), _f32)] * GSLOT
            + [pltpu.VMEM((EC, H), _f32)] * GSLOT
            + [pltpu.VMEM((EC,), _f32)] * GSLOT
            + [pltpu.SemaphoreType.DMA] * GSLOT
            + [pltpu.SemaphoreType.DMA] * 2
        ),
        compiler_params=_sc_cp,
    )

    scatter = pl.kernel(
        _scatter_sc,
        out_type=jax.ShapeDtypeStruct((NC, NPAD, H), _f32),
        mesh=_sc_mesh,
        scratch_types=(
            [pltpu.VMEM((ZR, H), _f32), pltpu.VMEM_SHARED((NPAD, H), _f32)]
            + [pltpu.VMEM((EC,), jnp.int32)] * SSLOT
            + [pltpu.VMEM((EC, H), _f32)] * SSLOT
            + [pltpu.SemaphoreType.DMA] * SSLOT
        ),
    )

    w1dmat = jnp.tile(W1[2 * H].reshape(1, H), (H, 1))
    wimat = jnp.tile(Wi.reshape(H, 1), (1, H))
    mi_parts = []
    for sidx in range(SLABS):
        src_s = lax.slice_in_dim(src, sidx * ES, (sidx + 1) * ES)
        dst_s = lax.slice_in_dim(dst, sidx * ES, (sidx + 1) * ES)
        g, d2 = gather(t1, t2, src_s, dst_s, px, py)
        msg = pl.pallas_call(
            _edge_body,
            grid=(ES // EB,),
            in_specs=[
                pl.BlockSpec((EB, H), lambda i: (i, 0)),
                pl.BlockSpec((1, EB // H, H), lambda i: (i, 0, 0)),
                _full((H, H)), _full((H, H)), _full((1, H)), _full((H, H)), _full((1, 1)),
            ],
            out_specs=pl.BlockSpec((EB, H), lambda i: (i, 0)),
            out_shape=jax.ShapeDtypeStruct((ES, H), _f32),
        )(g, d2.reshape(ES // EB, EB // H, H),
          w1dmat, W2, b2.reshape(1, H), wimat, bi.reshape(1, 1))
        mi_parts.append(scatter(msg, src_s))

    out = pl.pallas_call(
        _node_body,
        grid=(N // RB,),
        in_specs=(
            [pl.BlockSpec((RB, H), lambda i: (i, 0))]
            + [pl.BlockSpec((NC, RB, H), lambda i: (0, i, 0))] * SLABS
            + [_full((H, H)), _full((H, H)), _full((1, H)),
               _full((H, H)), _full((1, H)),
               _full((H, H)), _full((1, H)), _full((1, H)), _full((1, 1))]
        ),
        out_specs=pl.BlockSpec((RB, 1), lambda i: (i, 0)),
        out_shape=jax.ShapeDtypeStruct((N, 1), _f32),
    )(h, *mi_parts, Wh1[:H], Wh1[H:], bh1.reshape(1, H),
      Wh2, bh2.reshape(1, H), Wl1, bl1.reshape(1, H), Wl2.reshape(1, H),
      bl2.reshape(1, 1))

    return out
